# baseline (speedup 1.0000x reference)
import functools

import jax
import jax.numpy as jnp
from jax import lax
from jax.experimental import pallas as pl
from jax.experimental.pallas import tpu as pltpu
from jax.experimental.pallas import tpu_sc as plsc

_BATCH = 16384
_NUM_GENS = 1000
_TBL_PAD = 1024
_LANES = 16


@functools.cache
def _build(num_cores, num_subcores, batch):
    n_workers = num_cores * num_subcores
    chunk = batch // n_workers
    mesh = plsc.VectorSubcoreMesh(
        core_axis_name="c", subcore_axis_name="s", num_cores=num_cores)

    @functools.partial(
        pl.kernel,
        mesh=mesh,
        out_type=jax.ShapeDtypeStruct((batch,), jnp.float32),
        compiler_params=pltpu.CompilerParams(needs_layout_passes=False),
        scratch_types=[
            pltpu.VMEM((_NUM_GENS,), jnp.float32),
            pltpu.VMEM((chunk,), jnp.int32),
            pltpu.VMEM((chunk,), jnp.float32),
            pltpu.VMEM((chunk,), jnp.float32),
            pltpu.VMEM((chunk,), jnp.float32),
            pltpu.VMEM((2 * _LANES,), jnp.float32),
            pltpu.SemaphoreType.DMA,
        ],
    )
    def k(tbl_hbm, idx_hbm, clk_hbm, tdp_hbm, cd_hbm, out_hbm,
          tbl_v, idx_v, clk_v, tdp_v, out_v, cd_v, sem):
        wid = lax.axis_index("s") * num_cores + lax.axis_index("c")
        base = wid * chunk
        cp0 = pltpu.async_copy(tbl_hbm, tbl_v, sem)
        cp1 = pltpu.async_copy(idx_hbm.at[pl.ds(base, chunk)], idx_v, sem)
        cp2 = pltpu.async_copy(clk_hbm.at[pl.ds(base, chunk)], clk_v, sem)
        cp3 = pltpu.async_copy(tdp_hbm.at[pl.ds(base, chunk)], tdp_v, sem)
        cp4 = pltpu.async_copy(cd_hbm, cd_v, sem)
        cp0.wait()
        cp1.wait()
        cp2.wait()
        cp3.wait()
        cp4.wait()
        cc = cd_v[pl.ds(0, _LANES)]
        dd = cd_v[pl.ds(_LANES, _LANES)]
        for j in range(chunk // _LANES):
            sl = pl.ds(j * _LANES, _LANES)
            vals = plsc.load_gather(tbl_v, [idx_v[sl]])
            out_v[sl] = vals + cc * clk_v[sl] + dd * tdp_v[sl]
        pltpu.sync_copy(out_v, out_hbm.at[pl.ds(base, chunk)])

    return k


def kernel(x_gen, x_ix, x_max_clock_speed, x_max_tdp, gen_map, b, c, d):
    info = plsc.get_sparse_core_info()
    cd = jnp.concatenate([
        jnp.full((_LANES,), c, jnp.float32),
        jnp.full((_LANES,), d, jnp.float32),
    ])
    k = _build(1, info.num_subcores, _BATCH)
    return k(gen_map, x_gen, x_max_clock_speed, x_max_tdp, cd)
